# Initial kernel scaffold; baseline (speedup 1.0000x reference)
import jax
import jax.numpy as jnp
from jax.experimental import pallas as pl


def kernel(x, table):
    raise NotImplementedError("write your pallas kernel here")


# speedup vs baseline: 2.9766x; 2.9766x over previous
import jax
import jax.numpy as jnp
from jax import lax
from jax.experimental import pallas as pl
from jax.experimental.pallas import tpu as pltpu
from jax.experimental.pallas import tpu_sc as plsc

D_MODEL = 128
CHUNK = 128


def _gather_body(table_hbm, idx_hbm, out_hbm, idx_v, rows_v, gsem):
    num_cores = 2
    wid = lax.axis_index("s") * num_cores + lax.axis_index("c")
    n_chunks = idx_v.shape[0]
    pltpu.sync_copy(idx_hbm.at[wid], idx_v)

    def body(c, carry):
        pltpu.async_copy(table_hbm.at[idx_v.at[c]], rows_v, gsem).wait()
        base = (wid * n_chunks + c) * CHUNK
        pltpu.sync_copy(rows_v, out_hbm.at[pl.ds(base, CHUNK)])
        return carry

    lax.fori_loop(0, n_chunks, body, 0)


def kernel(x, table):
    batch, hist = x.shape
    b_total = batch * hist
    info = plsc.get_sparse_core_info()
    nw = info.num_cores * info.num_subcores
    b_per_w = b_total // nw
    n_chunks = b_per_w // CHUNK
    idx = x.reshape(nw, n_chunks, CHUNK)

    mesh = plsc.VectorSubcoreMesh(core_axis_name="c", subcore_axis_name="s")
    run = pl.kernel(
        _gather_body,
        out_type=jax.ShapeDtypeStruct((b_total, D_MODEL), jnp.float32),
        mesh=mesh,
        scratch_types=[
            pltpu.VMEM((n_chunks, CHUNK), jnp.int32),
            pltpu.VMEM((CHUNK, D_MODEL), jnp.float32),
            pltpu.SemaphoreType.DMA,
        ],
    )
    out = run(table, idx)
    return out.reshape(batch, hist, D_MODEL)


# speedup vs baseline: 3.3348x; 1.1203x over previous
import jax
import jax.numpy as jnp
from jax import lax
from jax.experimental import pallas as pl
from jax.experimental.pallas import tpu as pltpu
from jax.experimental.pallas import tpu_sc as plsc

D_MODEL = 128
CHUNK = 80
NBUF = 4


def _gather_body(table_hbm, idx_hbm, out_hbm, idx_v, rows_v, gsem, wsem):
    num_cores = 2
    wid = lax.axis_index("s") * num_cores + lax.axis_index("c")
    n_chunks = idx_v.shape[0]
    out_base = wid * n_chunks * CHUNK
    pltpu.sync_copy(idx_hbm.at[wid], idx_v)

    def start_gather(c, b):
        pltpu.async_copy(table_hbm.at[idx_v.at[c]], rows_v.at[b], gsem.at[b])

    def start_write(c, b):
        pltpu.async_copy(
            rows_v.at[b], out_hbm.at[pl.ds(out_base + c * CHUNK, CHUNK)],
            wsem.at[b])

    def wait_gather(b):
        pltpu.make_async_copy(
            table_hbm.at[pl.ds(0, CHUNK)], rows_v.at[b], gsem.at[b]).wait()

    def wait_write(b):
        pltpu.make_async_copy(
            rows_v.at[b], out_hbm.at[pl.ds(0, CHUNK)], wsem.at[b]).wait()

    start_gather(0, 0)
    start_gather(1, 1)
    start_gather(2, 2)
    wait_gather(0)
    start_write(0, 0)
    start_gather(3, 3)
    wait_gather(1)
    start_write(1, 1)

    def body(g, carry):
        c0 = NBUF + g * NBUF
        for j in range(NBUF):
            c = c0 + j
            wait_write(j)
            start_gather(c, j)
            d = (j + 2) % NBUF
            wait_gather(d)
            start_write(c - 2, d)
        return carry

    lax.fori_loop(0, (n_chunks - NBUF) // NBUF, body, 0)

    b_last = (n_chunks - 2) % NBUF
    wait_gather(b_last)
    start_write(n_chunks - 2, b_last)
    b_last = (n_chunks - 1) % NBUF
    wait_gather(b_last)
    start_write(n_chunks - 1, b_last)
    for b in range(NBUF):
        wait_write(b)


def kernel(x, table):
    batch, hist = x.shape
    b_total = batch * hist
    info = plsc.get_sparse_core_info()
    nw = info.num_cores * info.num_subcores
    b_per_w = b_total // nw
    n_chunks = b_per_w // CHUNK
    idx = x.reshape(nw, n_chunks, CHUNK)

    mesh = plsc.VectorSubcoreMesh(core_axis_name="c", subcore_axis_name="s")
    run = pl.kernel(
        _gather_body,
        out_type=jax.ShapeDtypeStruct((b_total, D_MODEL), jnp.float32),
        mesh=mesh,
        scratch_types=[
            pltpu.VMEM((n_chunks, CHUNK), jnp.int32),
            pltpu.VMEM((NBUF, CHUNK, D_MODEL), jnp.float32),
            pltpu.SemaphoreType.DMA((NBUF,)),
            pltpu.SemaphoreType.DMA((NBUF,)),
        ],
    )
    out = run(table, idx)
    return out.reshape(batch, hist, D_MODEL)


# speedup vs baseline: 3.3446x; 1.0030x over previous
import jax
import jax.numpy as jnp
from jax import lax
from jax.experimental import pallas as pl
from jax.experimental.pallas import tpu as pltpu
from jax.experimental.pallas import tpu_sc as plsc

D_MODEL = 128
CHUNK = 128
NBUF = 4
LAG = 2


def _gather_body(table_hbm, idx_hbm, out_hbm, idx_v, rows_v, gsem, wsem):
    num_cores = 2
    wid = lax.axis_index("s") * num_cores + lax.axis_index("c")
    n_chunks = idx_v.shape[0]
    out_base = wid * n_chunks * CHUNK
    pltpu.sync_copy(idx_hbm.at[wid], idx_v)

    def start_gather(c, b):
        pltpu.async_copy(table_hbm.at[idx_v.at[c]], rows_v.at[b], gsem.at[b])

    def start_write(c, b):
        pltpu.async_copy(
            rows_v.at[b], out_hbm.at[pl.ds(out_base + c * CHUNK, CHUNK)],
            wsem.at[b])

    def wait_gather(b):
        pltpu.make_async_copy(
            table_hbm.at[pl.ds(0, CHUNK)], rows_v.at[b], gsem.at[b]).wait()

    def wait_write(b):
        pltpu.make_async_copy(
            rows_v.at[b], out_hbm.at[pl.ds(0, CHUNK)], wsem.at[b]).wait()

    def step(c, b):
        if c_is_static := isinstance(c, int):
            assert b == c % NBUF
        if not c_is_static or c >= NBUF:
            wait_write(b)
        start_gather(c, b)
        d = (b - LAG) % NBUF
        if not c_is_static or c >= LAG:
            wait_gather(d)
            start_write(c - LAG, d)

    for c in range(NBUF):
        step(c, c % NBUF)

    n_main = (n_chunks - NBUF) // NBUF * NBUF
    def body(g, carry):
        c0 = NBUF + g * NBUF
        for j in range(NBUF):
            step(c0 + j, j)
        return carry
    lax.fori_loop(0, n_main // NBUF, body, 0)

    for c in range(NBUF + n_main, n_chunks):
        step(c, c % NBUF)

    for c in range(n_chunks - LAG, n_chunks):
        b = c % NBUF
        wait_gather(b)
        start_write(c, b)
    for b in range(NBUF):
        wait_write(b)


def kernel(x, table):
    batch, hist = x.shape
    b_total = batch * hist
    info = plsc.get_sparse_core_info()
    nw = info.num_cores * info.num_subcores
    b_per_w = b_total // nw
    n_chunks = b_per_w // CHUNK
    idx = x.reshape(nw, n_chunks, CHUNK)

    mesh = plsc.VectorSubcoreMesh(core_axis_name="c", subcore_axis_name="s")
    run = pl.kernel(
        _gather_body,
        out_type=jax.ShapeDtypeStruct((b_total, D_MODEL), jnp.float32),
        mesh=mesh,
        scratch_types=[
            pltpu.VMEM((n_chunks, CHUNK), jnp.int32),
            pltpu.VMEM((NBUF, CHUNK, D_MODEL), jnp.float32),
            pltpu.SemaphoreType.DMA((NBUF,)),
            pltpu.SemaphoreType.DMA((NBUF,)),
        ],
    )
    out = run(table, idx)
    return out.reshape(batch, hist, D_MODEL)


# speedup vs baseline: 3.3506x; 1.0018x over previous
import jax
import jax.numpy as jnp
from jax import lax
from jax.experimental import pallas as pl
from jax.experimental.pallas import tpu as pltpu
from jax.experimental.pallas import tpu_sc as plsc

D_MODEL = 128
CHUNK = 128
NBUF = 6
LAG = 3


def _gather_body(table_hbm, idx_hbm, out_hbm, idx_v, rows_v, gsem, wsem):
    num_cores = 2
    wid = lax.axis_index("s") * num_cores + lax.axis_index("c")
    n_chunks = idx_v.shape[0]
    out_base = wid * n_chunks * CHUNK
    pltpu.sync_copy(idx_hbm.at[wid], idx_v)

    def start_gather(c, b):
        pltpu.async_copy(table_hbm.at[idx_v.at[c]], rows_v.at[b], gsem.at[b])

    def start_write(c, b):
        pltpu.async_copy(
            rows_v.at[b], out_hbm.at[pl.ds(out_base + c * CHUNK, CHUNK)],
            wsem.at[b])

    def wait_gather(b):
        pltpu.make_async_copy(
            table_hbm.at[pl.ds(0, CHUNK)], rows_v.at[b], gsem.at[b]).wait()

    def wait_write(b):
        pltpu.make_async_copy(
            rows_v.at[b], out_hbm.at[pl.ds(0, CHUNK)], wsem.at[b]).wait()

    def step(c, b):
        if c_is_static := isinstance(c, int):
            assert b == c % NBUF
        if not c_is_static or c >= NBUF:
            wait_write(b)
        start_gather(c, b)
        d = (b - LAG) % NBUF
        if not c_is_static or c >= LAG:
            wait_gather(d)
            start_write(c - LAG, d)

    for c in range(NBUF):
        step(c, c % NBUF)

    n_main = (n_chunks - NBUF) // NBUF * NBUF
    def body(g, carry):
        c0 = NBUF + g * NBUF
        for j in range(NBUF):
            step(c0 + j, j)
        return carry
    lax.fori_loop(0, n_main // NBUF, body, 0)

    for c in range(NBUF + n_main, n_chunks):
        step(c, c % NBUF)

    for c in range(n_chunks - LAG, n_chunks):
        b = c % NBUF
        wait_gather(b)
        start_write(c, b)
    for b in range(NBUF):
        wait_write(b)


def kernel(x, table):
    batch, hist = x.shape
    b_total = batch * hist
    info = plsc.get_sparse_core_info()
    nw = info.num_cores * info.num_subcores
    b_per_w = b_total // nw
    n_chunks = b_per_w // CHUNK
    idx = x.reshape(nw, n_chunks, CHUNK)

    mesh = plsc.VectorSubcoreMesh(core_axis_name="c", subcore_axis_name="s")
    run = pl.kernel(
        _gather_body,
        out_type=jax.ShapeDtypeStruct((b_total, D_MODEL), jnp.float32),
        mesh=mesh,
        scratch_types=[
            pltpu.VMEM((n_chunks, CHUNK), jnp.int32),
            pltpu.VMEM((NBUF, CHUNK, D_MODEL), jnp.float32),
            pltpu.SemaphoreType.DMA((NBUF,)),
            pltpu.SemaphoreType.DMA((NBUF,)),
        ],
    )
    out = run(table, idx)
    return out.reshape(batch, hist, D_MODEL)
